# Initial kernel scaffold; baseline (speedup 1.0000x reference)
import jax
import jax.numpy as jnp
from jax.experimental import pallas as pl


def kernel(x, pe):
    raise NotImplementedError("write your pallas kernel here")


# speedup vs baseline: 1.4938x; 1.4938x over previous
import jax
import jax.numpy as jnp
from jax.experimental import pallas as pl


def _add_kernel(x_ref, pe_ref, o_ref):
    o_ref[...] = x_ref[...] + pe_ref[...]


def kernel(x, pe):
    B, T, D = x.shape
    BS = 512
    grid = (T // BS, B)
    return pl.pallas_call(
        _add_kernel,
        grid=grid,
        in_specs=[
            pl.BlockSpec((1, BS, D), lambda i, j: (j, i, 0)),
            pl.BlockSpec((BS, D), lambda i, j: (i, 0)),
        ],
        out_specs=pl.BlockSpec((1, BS, D), lambda i, j: (j, i, 0)),
        out_shape=jax.ShapeDtypeStruct((B, T, D), x.dtype),
    )(x, pe[:T])


# speedup vs baseline: 1.6678x; 1.1165x over previous
import jax
import jax.numpy as jnp
from jax.experimental import pallas as pl


def _add_kernel(x_ref, pe_ref, o_ref):
    o_ref[...] = x_ref[...] + pe_ref[...]


def kernel(x, pe):
    B, T, D = x.shape
    BS = 1024
    grid = (T // BS, B)
    return pl.pallas_call(
        _add_kernel,
        grid=grid,
        in_specs=[
            pl.BlockSpec((1, BS, D), lambda i, j: (j, i, 0)),
            pl.BlockSpec((BS, D), lambda i, j: (i, 0)),
        ],
        out_specs=pl.BlockSpec((1, BS, D), lambda i, j: (j, i, 0)),
        out_shape=jax.ShapeDtypeStruct((B, T, D), x.dtype),
    )(x, pe[:T])


# speedup vs baseline: 1.7351x; 1.0404x over previous
import jax
import jax.numpy as jnp
from jax.experimental import pallas as pl


def _add_kernel(x_ref, pe_ref, o_ref):
    o_ref[...] = x_ref[...] + pe_ref[...]


def kernel(x, pe):
    B, T, D = x.shape
    BS = 2048
    grid = (T // BS, B)
    return pl.pallas_call(
        _add_kernel,
        grid=grid,
        in_specs=[
            pl.BlockSpec((1, BS, D), lambda i, j: (j, i, 0)),
            pl.BlockSpec((BS, D), lambda i, j: (i, 0)),
        ],
        out_specs=pl.BlockSpec((1, BS, D), lambda i, j: (j, i, 0)),
        out_shape=jax.ShapeDtypeStruct((B, T, D), x.dtype),
    )(x, pe[:T])


# speedup vs baseline: 1.7369x; 1.0010x over previous
import jax
import jax.numpy as jnp
from jax.experimental import pallas as pl
from jax.experimental.pallas import tpu as pltpu


def _add_kernel(x_ref, pe_ref, o_ref):
    o_ref[...] = x_ref[...] + pe_ref[...]


def kernel(x, pe):
    B, T, D = x.shape
    BS = 2048
    grid = (T // BS, B)
    return pl.pallas_call(
        _add_kernel,
        grid=grid,
        in_specs=[
            pl.BlockSpec((1, BS, D), lambda i, j: (j, i, 0)),
            pl.BlockSpec((BS, D), lambda i, j: (i, 0)),
        ],
        out_specs=pl.BlockSpec((1, BS, D), lambda i, j: (j, i, 0)),
        out_shape=jax.ShapeDtypeStruct((B, T, D), x.dtype),
        compiler_params=pltpu.CompilerParams(
            dimension_semantics=("parallel", "parallel"),
        ),
    )(x, pe[:T])
